# Initial kernel scaffold; baseline (speedup 1.0000x reference)
import jax
import jax.numpy as jnp
from jax.experimental import pallas as pl


def kernel(X, grid, grid_norm):
    raise NotImplementedError("write your pallas kernel here")


# speedup vs baseline: 4.8342x; 4.8342x over previous
import functools

import jax
import jax.numpy as jnp
from jax import lax
from jax.experimental import pallas as pl
from jax.experimental.pallas import tpu as pltpu
from jax.experimental.pallas import tpu_sc as plsc

_N = 524288
_K = 8
_C = 256
_B = 8192
_NB = _N // _B

_NC = 2
_NS = 16
_NW = _NC * _NS
_ROWS_PER_W = _N // _NW
_CHUNK = 4096


def _score_argmax_body(xt_ref, g2_ref, norm_ref, idx_ref):
    sc = jnp.dot(g2_ref[...], xt_ref[...], preferred_element_type=jnp.float32)
    sc = sc - norm_ref[...]
    m = jnp.max(sc, axis=0, keepdims=True)
    ii = lax.broadcasted_iota(jnp.int32, (_C, _B), 0)
    cand = jnp.where(sc == m, ii, _C)
    idx_ref[0] = jnp.min(cand, axis=0, keepdims=True)


def _tc_score_argmax(xt, g2, norm2):
    return pl.pallas_call(
        _score_argmax_body,
        grid=(_NB,),
        in_specs=[
            pl.BlockSpec((_K, _B), lambda i: (0, i)),
            pl.BlockSpec((_C, _K), lambda i: (0, 0)),
            pl.BlockSpec((_C, 1), lambda i: (0, 0)),
        ],
        out_specs=pl.BlockSpec((1, 1, _B), lambda i: (i, 0, 0)),
        out_shape=jax.ShapeDtypeStruct((_NB, 1, _B), jnp.int32),
    )(xt, g2, norm2)


def _sc_gather(idx_flat, table):
    mesh = plsc.VectorSubcoreMesh(core_axis_name="c", subcore_axis_name="s")

    @functools.partial(
        pl.kernel,
        mesh=mesh,
        out_type=jax.ShapeDtypeStruct((_N * _K,), jnp.float32),
        scratch_types=[
            pltpu.VMEM((_ROWS_PER_W,), jnp.int32),
            pltpu.VMEM((_C * _K,), jnp.float32),
            pltpu.VMEM((_CHUNK * _K,), jnp.float32),
        ],
        compiler_params=pltpu.CompilerParams(needs_layout_passes=False),
    )
    def k(idx_hbm, table_hbm, out_hbm, idx_v, table_v, rows_v):
        wid = lax.axis_index("s") * _NC + lax.axis_index("c")
        row_base = wid * _ROWS_PER_W
        pltpu.sync_copy(table_hbm, table_v)
        pltpu.sync_copy(
            idx_hbm.at[pl.ds(pl.multiple_of(row_base, 128), _ROWS_PER_W)],
            idx_v)

        pos0 = lax.iota(jnp.int32, 16) * _K

        def chunk(c, carry):
            def body16(c2, carry2):
                coff = pl.multiple_of(c * _CHUNK + c2 * 16, 8)
                rvec = idx_v[pl.ds(coff, 16)] * _K
                pos = pos0 + c2 * (16 * _K)
                for kk in range(_K):
                    vals = plsc.load_gather(table_v, [rvec + kk])
                    plsc.store_scatter(rows_v, [pos + kk], vals)
                return carry2

            lax.fori_loop(0, _CHUNK // 16, body16, 0, unroll=2)
            ooff = pl.multiple_of((row_base + c * _CHUNK) * _K, 128)
            pltpu.sync_copy(
                rows_v.at[pl.ds(0, _CHUNK * _K)],
                out_hbm.at[pl.ds(ooff, _CHUNK * _K)])
            return carry

        lax.fori_loop(0, _ROWS_PER_W // _CHUNK, chunk, 0)

    return k(idx_flat, table)


def kernel(X, grid, grid_norm):
    xt = X.T
    g2 = 2.0 * grid
    norm2 = grid_norm.reshape(_C, 1)
    idx3 = _tc_score_argmax(xt, g2, norm2)
    idx_flat = idx3.reshape(_N)
    quantized = _sc_gather(idx_flat, grid.reshape(_C * _K)).reshape(_N, _K)
    return (quantized, idx_flat.astype(jnp.uint8))
